# baseline (speedup 1.0000x reference)
import functools

import jax
import jax.numpy as jnp
from jax import lax
from jax.experimental import pallas as pl
from jax.experimental.pallas import tpu as pltpu
from jax.experimental.pallas import tpu_sc as plsc

N = 10000
E = 320000
D = 128
H = 128
C = 16

NTILES = 16
NP = 10240
ROWS_PER_TILE = NP // NTILES
CHUNK = 128
EP = 327680
ECHUNKS = 2 * EP // CHUNK
TILE_ECHUNKS = EP // (NTILES * CHUNK)
BLK = 16
NBLK = TILE_ECHUNKS // BLK

_sc_mesh = plsc.VectorSubcoreMesh(core_axis_name="c", subcore_axis_name="s")


@functools.partial(
    pl.kernel,
    out_type=jax.ShapeDtypeStruct((2 * NP,), jnp.float32),
    mesh=_sc_mesh,
    scratch_types=[
        pltpu.VMEM((CHUNK,), jnp.float32),
        pltpu.VMEM((BLK, CHUNK), jnp.int32),
        pltpu.VMEM((ROWS_PER_TILE,), jnp.float32),
        pltpu.VMEM_SHARED((NP,), jnp.float32),
        pltpu.SemaphoreType.DMA,
    ],
)
def _deg_kernel(dst_hbm, deg_hbm, ones_v, dblk, stage_v, acc_sh, dsem):
    c = lax.axis_index("c")
    s = lax.axis_index("s")
    for j in range(CHUNK // 16):
        ones_v[pl.ds(j * 16, 16)] = jnp.ones((16,), jnp.float32)
    for j in range(ROWS_PER_TILE // 16):
        stage_v[pl.ds(j * 16, 16)] = jnp.zeros((16,), jnp.float32)
    pltpu.sync_copy(stage_v, acc_sh.at[pl.ds(s * ROWS_PER_TILE, ROWS_PER_TILE)])
    plsc.subcore_barrier()

    r0 = c * (ECHUNKS // 2) + s * TILE_ECHUNKS

    def body(i, carry):
        pltpu.sync_copy(dst_hbm.at[pl.ds(r0 + i * BLK, BLK)], dblk)
        descs = [pltpu.async_copy(ones_v, acc_sh.at[dblk.at[j]], dsem,
                                  add=True) for j in range(BLK)]
        for d in descs:
            d.wait()
        return carry

    lax.fori_loop(0, NBLK, body, 0)
    plsc.subcore_barrier()
    pltpu.sync_copy(acc_sh.at[pl.ds(s * ROWS_PER_TILE, ROWS_PER_TILE)], stage_v)
    pltpu.sync_copy(stage_v,
                    deg_hbm.at[pl.ds(c * NP + s * ROWS_PER_TILE, ROWS_PER_TILE)])


def _make_scatter(W):
    nstage = ROWS_PER_TILE // CHUNK

    @functools.partial(
        pl.kernel,
        out_type=jax.ShapeDtypeStruct((2, NP, W), jnp.float32),
        mesh=_sc_mesh,
        scratch_types=[
            pltpu.VMEM((BLK, CHUNK), jnp.int32),
            pltpu.VMEM((BLK, CHUNK), jnp.int32),
            pltpu.VMEM((2, CHUNK, W), jnp.float32),
            pltpu.VMEM_SHARED((NP, W), jnp.float32),
            pltpu.SemaphoreType.DMA,
            pltpu.SemaphoreType.DMA,
        ],
    )
    def scat(hp_hbm, src_hbm, dst_hbm, zeros_hbm, agg_hbm,
             sblk, dblk, rows, acc, gsem, ssem):
        c = lax.axis_index("c")
        s = lax.axis_index("s")
        row0 = s * ROWS_PER_TILE
        pltpu.sync_copy(zeros_hbm, rows.at[0])
        for j in range(nstage):
            pltpu.sync_copy(rows.at[0], acc.at[pl.ds(row0 + j * CHUNK, CHUNK)])
        plsc.subcore_barrier()

        r0 = c * (ECHUNKS // 2) + s * TILE_ECHUNKS

        def body(i, carry):
            pltpu.sync_copy(src_hbm.at[pl.ds(r0 + i * BLK, BLK)], sblk)
            pltpu.sync_copy(dst_hbm.at[pl.ds(r0 + i * BLK, BLK)], dblk)
            desc = pltpu.async_copy(hp_hbm.at[sblk.at[0]], rows.at[0], gsem)
            for j in range(BLK):
                b = j & 1
                if j + 1 < BLK:
                    nxt = pltpu.async_copy(hp_hbm.at[sblk.at[j + 1]],
                                           rows.at[b ^ 1], gsem)
                desc.wait()
                pltpu.sync_copy(rows.at[b], acc.at[dblk.at[j]], add=True)
                if j + 1 < BLK:
                    desc = nxt
            return carry

        lax.fori_loop(0, NBLK, body, 0)
        plsc.subcore_barrier()
        for j in range(nstage):
            r = row0 + j * CHUNK
            pltpu.sync_copy(acc.at[pl.ds(r, CHUNK)], rows.at[0])
            pltpu.sync_copy(rows.at[0], agg_hbm.at[c, pl.ds(r, CHUNK)])

    return scat


_scatter128 = _make_scatter(H)


def _mm_body(xl_ref, xg_ref, w_ref, h_ref):
    h_ref[0] = jnp.dot(xl_ref[...], w_ref[0], preferred_element_type=jnp.float32)
    h_ref[1] = jnp.dot(xg_ref[...], w_ref[1], preferred_element_type=jnp.float32)


def _prep_body(h_ref, degc_ref, hp_ref, dinv_ref):
    for g in range(2):
        d = lax.rsqrt(degc_ref[g] + 1.0)
        dinv_ref[g] = d
        hp_ref[g] = d * h_ref[g]


def _stage_body(agg_ref, hp_ref, d_ref, b_ref, w_ref, out_ref):
    for g in range(2):
        d = d_ref[g]
        a = d * (agg_ref[g, :N] + hp_ref[g]) + b_ref[g]
        a = jnp.maximum(a, 0.0)
        out_ref[g] = d * jnp.dot(a, w_ref[g],
                                 preferred_element_type=jnp.float32)


def _stage3_body(agg_ref, hp_ref, d_ref, b_ref, out_ref):
    for g in range(2):
        d = d_ref[g]
        a = d * (agg_ref[g, :N] + hp_ref[g]) + b_ref[g]
        out_ref[g] = d * jnp.maximum(a, 0.0)


def _final_body(agg_ref, hq_ref, d_ref, b_ref, w_ref, out_ref):
    for g in range(2):
        a = agg_ref[g, :N] + hq_ref[g]
        out_ref[g] = d_ref[g] * jnp.dot(
            a, w_ref[g], preferred_element_type=jnp.float32) + b_ref[g]


def _tc(body, out_shape, *args):
    return pl.pallas_call(body, out_shape=out_shape)(*args)


def kernel(Lnc_f_features, Gene_f_features, Lnc_f_edge_index,
           Gene_f_edge_index, W1l, b1l, W2l, b2l, W3l, b3l,
           W1g, b1g, W2g, b2g, W3g, b3g):
    f32 = jnp.float32
    npad = EP - E
    pad_src = (jnp.arange(npad, dtype=jnp.int32) * 131) % N
    pad_dst = N + (jnp.arange(npad, dtype=jnp.int32) % (NP - N))
    src = jnp.concatenate([Lnc_f_edge_index[0], pad_src,
                           Gene_f_edge_index[0] + N, pad_src + N])
    src = src.reshape(ECHUNKS, CHUNK)
    dst = jnp.concatenate([Lnc_f_edge_index[1], pad_dst,
                           Gene_f_edge_index[1], pad_dst])
    dst = dst.reshape(ECHUNKS, CHUNK)
    W1 = jnp.stack([W1l, W1g])
    W2 = jnp.stack([W2l, W2g])
    W3 = jnp.stack([W3l, W3g])
    b1 = jnp.stack([b1l, b1g])[:, None, :]
    b2 = jnp.stack([b2l, b2g])[:, None, :]
    b3 = jnp.stack([b3l, b3g])[:, None, :]
    zeros_h = jnp.zeros((CHUNK, H), f32)

    deg = _deg_kernel(dst).reshape(2, NP)
    degc = deg[:, :N, None]
    h1 = _tc(_mm_body, jax.ShapeDtypeStruct((2, N, H), f32),
             Lnc_f_features, Gene_f_features, W1)
    hp1, dinvc = _tc(_prep_body,
                     (jax.ShapeDtypeStruct((2, N, H), f32),
                      jax.ShapeDtypeStruct((2, N, 1), f32)),
                     h1, degc)
    agg1 = _scatter128(hp1.reshape(2 * N, H), src, dst, zeros_h)
    hp2 = _tc(_stage_body, jax.ShapeDtypeStruct((2, N, H), f32),
              agg1, hp1, dinvc, b1, W2)
    agg2 = _scatter128(hp2.reshape(2 * N, H), src, dst, zeros_h)
    hq = _tc(_stage3_body, jax.ShapeDtypeStruct((2, N, H), f32),
             agg2, hp2, dinvc, b2)
    agg3 = _scatter128(hq.reshape(2 * N, H), src, dst, zeros_h)
    out = _tc(_final_body, jax.ShapeDtypeStruct((2, N, C), f32),
              agg3, hq, dinvc, b3, W3)
    return out[0], out[1]


# speedup vs baseline: 1.0141x; 1.0141x over previous
import functools

import jax
import numpy as np
import jax.numpy as jnp
from jax import lax
from jax.experimental import pallas as pl
from jax.experimental.pallas import tpu as pltpu
from jax.experimental.pallas import tpu_sc as plsc

N = 10000
E = 320000
D = 128
H = 128
C = 16

NTILES = 16
NP = 10240
ROWS_PER_TILE = NP // NTILES
CHUNK = 128
EP = 327680
ECHUNKS = 2 * EP // CHUNK
TILE_ECHUNKS = EP // (NTILES * CHUNK)
BLK = 16
NBLK = TILE_ECHUNKS // BLK

_sc_mesh = plsc.VectorSubcoreMesh(core_axis_name="c", subcore_axis_name="s")


@functools.partial(
    pl.kernel,
    out_type=jax.ShapeDtypeStruct((2 * NP,), jnp.float32),
    mesh=_sc_mesh,
    scratch_types=[
        pltpu.VMEM((CHUNK,), jnp.float32),
        pltpu.VMEM((BLK, CHUNK), jnp.int32),
        pltpu.VMEM((ROWS_PER_TILE,), jnp.float32),
        pltpu.VMEM_SHARED((NP,), jnp.float32),
        pltpu.SemaphoreType.DMA,
    ],
)
def _deg_kernel(dst_hbm, deg_hbm, ones_v, dblk, stage_v, acc_sh, dsem):
    c = lax.axis_index("c")
    s = lax.axis_index("s")
    for j in range(CHUNK // 16):
        ones_v[pl.ds(j * 16, 16)] = jnp.ones((16,), jnp.float32)
    for j in range(ROWS_PER_TILE // 16):
        stage_v[pl.ds(j * 16, 16)] = jnp.zeros((16,), jnp.float32)
    pltpu.sync_copy(stage_v, acc_sh.at[pl.ds(s * ROWS_PER_TILE, ROWS_PER_TILE)])
    plsc.subcore_barrier()

    r0 = c * (ECHUNKS // 2) + s * TILE_ECHUNKS

    def body(i, carry):
        pltpu.sync_copy(dst_hbm.at[pl.ds(r0 + i * BLK, BLK)], dblk)
        descs = [pltpu.async_copy(ones_v, acc_sh.at[dblk.at[j]], dsem,
                                  add=True) for j in range(BLK)]
        for d in descs:
            d.wait()
        return carry

    lax.fori_loop(0, NBLK, body, 0)
    plsc.subcore_barrier()
    pltpu.sync_copy(acc_sh.at[pl.ds(s * ROWS_PER_TILE, ROWS_PER_TILE)], stage_v)
    pltpu.sync_copy(stage_v,
                    deg_hbm.at[pl.ds(c * NP + s * ROWS_PER_TILE, ROWS_PER_TILE)])


def _make_scatter(W):
    nstage = ROWS_PER_TILE // CHUNK

    @functools.partial(
        pl.kernel,
        out_type=jax.ShapeDtypeStruct((2, NP, W), jnp.float32),
        mesh=_sc_mesh,
        scratch_types=[
            pltpu.VMEM((BLK, CHUNK), jnp.int32),
            pltpu.VMEM((BLK, CHUNK), jnp.int32),
            pltpu.VMEM((2, CHUNK, W), jnp.float32),
            pltpu.VMEM_SHARED((NP, W), jnp.float32),
            pltpu.SemaphoreType.DMA,
            pltpu.SemaphoreType.DMA,
        ],
    )
    def scat(hp_hbm, src_hbm, dst_hbm, zeros_hbm, agg_hbm,
             sblk, dblk, rows, acc, gsem, ssem):
        c = lax.axis_index("c")
        s = lax.axis_index("s")
        row0 = s * ROWS_PER_TILE
        pltpu.sync_copy(zeros_hbm, rows.at[0])
        for j in range(nstage):
            pltpu.sync_copy(rows.at[0], acc.at[pl.ds(row0 + j * CHUNK, CHUNK)])
        plsc.subcore_barrier()

        r0 = c * (ECHUNKS // 2) + s * TILE_ECHUNKS

        def body(i, carry):
            pltpu.sync_copy(src_hbm.at[pl.ds(r0 + i * BLK, BLK)], sblk)
            pltpu.sync_copy(dst_hbm.at[pl.ds(r0 + i * BLK, BLK)], dblk)
            desc = pltpu.async_copy(hp_hbm.at[c].at[sblk.at[0]],
                                    rows.at[0], gsem)
            for j in range(BLK):
                b = j & 1
                if j + 1 < BLK:
                    nxt = pltpu.async_copy(hp_hbm.at[c].at[sblk.at[j + 1]],
                                           rows.at[b ^ 1], gsem)
                desc.wait()
                pltpu.sync_copy(rows.at[b], acc.at[dblk.at[j]], add=True)
                if j + 1 < BLK:
                    desc = nxt
            return carry

        lax.fori_loop(0, NBLK, body, 0)
        plsc.subcore_barrier()
        for j in range(nstage):
            r = row0 + j * CHUNK
            pltpu.sync_copy(acc.at[pl.ds(r, CHUNK)], rows.at[0])
            pltpu.sync_copy(rows.at[0], agg_hbm.at[c, pl.ds(r, CHUNK)])

    return scat


_scatter128 = _make_scatter(H)


def _mmprep_body(xl_ref, xg_ref, w_ref, degc_ref, hp_ref, dinv_ref):
    for g, x_ref in ((0, xl_ref), (1, xg_ref)):
        d = lax.rsqrt(degc_ref[g] + 1.0)
        dinv_ref[g] = d
        hp_ref[g] = d * jnp.dot(x_ref[...], w_ref[g],
                                preferred_element_type=jnp.float32)


def _stage_body(agg_ref, hp_ref, d_ref, b_ref, w_ref, out_ref):
    for g in range(2):
        d = d_ref[g]
        a = d * (agg_ref[g, :N] + hp_ref[g]) + b_ref[g]
        a = jnp.maximum(a, 0.0)
        out_ref[g] = d * jnp.dot(a, w_ref[g],
                                 preferred_element_type=jnp.float32)


def _stage3_body(agg_ref, hp_ref, d_ref, b_ref, out_ref):
    for g in range(2):
        d = d_ref[g]
        a = d * (agg_ref[g, :N] + hp_ref[g]) + b_ref[g]
        out_ref[g] = d * jnp.maximum(a, 0.0)


def _final_body(agg_ref, hq_ref, d_ref, b_ref, w_ref, outl_ref, outg_ref):
    for g, o_ref in ((0, outl_ref), (1, outg_ref)):
        a = agg_ref[g, :N] + hq_ref[g]
        o_ref[...] = d_ref[g] * jnp.dot(
            a, w_ref[g], preferred_element_type=jnp.float32) + b_ref[g]


def _tc(body, out_shape, *args):
    return pl.pallas_call(body, out_shape=out_shape)(*args)


def kernel(Lnc_f_features, Gene_f_features, Lnc_f_edge_index,
           Gene_f_edge_index, W1l, b1l, W2l, b2l, W3l, b3l,
           W1g, b1g, W2g, b2g, W3g, b3g):
    f32 = jnp.float32
    npad = EP - E
    pad_src = np.asarray((np.arange(npad) * 131) % N, np.int32)
    pad_dst = np.asarray(N + np.arange(npad) % (NP - N), np.int32)
    src = jnp.concatenate([Lnc_f_edge_index[0], pad_src,
                           Gene_f_edge_index[0], pad_src])
    src = src.reshape(ECHUNKS, CHUNK)
    dst = jnp.concatenate([Lnc_f_edge_index[1], pad_dst,
                           Gene_f_edge_index[1], pad_dst])
    dst = dst.reshape(ECHUNKS, CHUNK)
    W1 = jnp.stack([W1l, W1g])
    W2 = jnp.stack([W2l, W2g])
    W3 = jnp.stack([W3l, W3g])
    b1 = jnp.stack([b1l, b1g])[:, None, :]
    b2 = jnp.stack([b2l, b2g])[:, None, :]
    b3 = jnp.stack([b3l, b3g])[:, None, :]
    zeros_h = jnp.zeros((CHUNK, H), f32)

    deg = _deg_kernel(dst).reshape(2, NP)
    degc = deg[:, :N, None]
    hp1, dinvc = _tc(_mmprep_body,
                     (jax.ShapeDtypeStruct((2, N, H), f32),
                      jax.ShapeDtypeStruct((2, N, 1), f32)),
                     Lnc_f_features, Gene_f_features, W1, degc)
    agg1 = _scatter128(hp1, src, dst, zeros_h)
    hp2 = _tc(_stage_body, jax.ShapeDtypeStruct((2, N, H), f32),
              agg1, hp1, dinvc, b1, W2)
    agg2 = _scatter128(hp2, src, dst, zeros_h)
    hq = _tc(_stage3_body, jax.ShapeDtypeStruct((2, N, H), f32),
             agg2, hp2, dinvc, b2)
    agg3 = _scatter128(hq, src, dst, zeros_h)
    return _tc(_final_body, (jax.ShapeDtypeStruct((N, C), f32),
                             jax.ShapeDtypeStruct((N, C), f32)),
               agg3, hq, dinvc, b3, W3)


# speedup vs baseline: 1.0532x; 1.0386x over previous
import functools

import jax
import numpy as np
import jax.numpy as jnp
from jax import lax
from jax.experimental import pallas as pl
from jax.experimental.pallas import tpu as pltpu
from jax.experimental.pallas import tpu_sc as plsc

N = 10000
E = 320000
D = 128
H = 128
C = 16

NTILES = 16
NP = 10240
ROWS_PER_TILE = NP // NTILES
CHUNK = 128
EP = 327680
ECHUNKS = 2 * EP // CHUNK
TILE_ECHUNKS = EP // (NTILES * CHUNK)
BLK = 32
NBLK = TILE_ECHUNKS // BLK

_sc_mesh = plsc.VectorSubcoreMesh(core_axis_name="c", subcore_axis_name="s")


@functools.partial(
    pl.kernel,
    out_type=jax.ShapeDtypeStruct((2 * NP,), jnp.float32),
    mesh=_sc_mesh,
    scratch_types=[
        pltpu.VMEM((CHUNK,), jnp.float32),
        pltpu.VMEM((BLK, CHUNK), jnp.int32),
        pltpu.VMEM((ROWS_PER_TILE,), jnp.float32),
        pltpu.VMEM_SHARED((NP,), jnp.float32),
        pltpu.SemaphoreType.DMA,
    ],
)
def _deg_kernel(dst_hbm, deg_hbm, ones_v, dblk, stage_v, acc_sh, dsem):
    c = lax.axis_index("c")
    s = lax.axis_index("s")
    for j in range(CHUNK // 16):
        ones_v[pl.ds(j * 16, 16)] = jnp.ones((16,), jnp.float32)
    for j in range(ROWS_PER_TILE // 16):
        stage_v[pl.ds(j * 16, 16)] = jnp.zeros((16,), jnp.float32)
    pltpu.sync_copy(stage_v, acc_sh.at[pl.ds(s * ROWS_PER_TILE, ROWS_PER_TILE)])
    plsc.subcore_barrier()

    r0 = c * (ECHUNKS // 2) + s * TILE_ECHUNKS

    def body(i, carry):
        pltpu.sync_copy(dst_hbm.at[pl.ds(r0 + i * BLK, BLK)], dblk)
        descs = [pltpu.async_copy(ones_v, acc_sh.at[dblk.at[j]], dsem,
                                  add=True) for j in range(BLK)]
        for d in descs:
            d.wait()
        return carry

    lax.fori_loop(0, NBLK, body, 0)
    plsc.subcore_barrier()
    pltpu.sync_copy(acc_sh.at[pl.ds(s * ROWS_PER_TILE, ROWS_PER_TILE)], stage_v)
    pltpu.sync_copy(stage_v,
                    deg_hbm.at[pl.ds(c * NP + s * ROWS_PER_TILE, ROWS_PER_TILE)])


def _make_scatter(W):
    nstage = ROWS_PER_TILE // CHUNK

    @functools.partial(
        pl.kernel,
        out_type=jax.ShapeDtypeStruct((2, NP, W), jnp.float32),
        mesh=_sc_mesh,
        scratch_types=[
            pltpu.VMEM((BLK, CHUNK), jnp.int32),
            pltpu.VMEM((BLK, CHUNK), jnp.int32),
            pltpu.VMEM((2, CHUNK, W), jnp.float32),
            pltpu.VMEM_SHARED((NP, W), jnp.float32),
            pltpu.SemaphoreType.DMA,
            pltpu.SemaphoreType.DMA,
        ],
    )
    def scat(hp_hbm, src_hbm, dst_hbm, zeros_hbm, agg_hbm,
             sblk, dblk, rows, acc, gsem, ssem):
        c = lax.axis_index("c")
        s = lax.axis_index("s")
        row0 = s * ROWS_PER_TILE
        pltpu.sync_copy(zeros_hbm, rows.at[0])
        for j in range(nstage):
            pltpu.sync_copy(rows.at[0], acc.at[pl.ds(row0 + j * CHUNK, CHUNK)])
        plsc.subcore_barrier()

        r0 = c * (ECHUNKS // 2) + s * TILE_ECHUNKS

        def body(i, carry):
            pltpu.sync_copy(src_hbm.at[pl.ds(r0 + i * BLK, BLK)], sblk)
            pltpu.sync_copy(dst_hbm.at[pl.ds(r0 + i * BLK, BLK)], dblk)
            desc = pltpu.async_copy(hp_hbm.at[c].at[sblk.at[0]],
                                    rows.at[0], gsem)
            for j in range(BLK):
                b = j & 1
                if j + 1 < BLK:
                    nxt = pltpu.async_copy(hp_hbm.at[c].at[sblk.at[j + 1]],
                                           rows.at[b ^ 1], gsem)
                desc.wait()
                pltpu.sync_copy(rows.at[b], acc.at[dblk.at[j]], add=True)
                if j + 1 < BLK:
                    desc = nxt
            return carry

        lax.fori_loop(0, NBLK, body, 0)
        plsc.subcore_barrier()
        for j in range(nstage):
            r = row0 + j * CHUNK
            pltpu.sync_copy(acc.at[pl.ds(r, CHUNK)], rows.at[0])
            pltpu.sync_copy(rows.at[0], agg_hbm.at[c, pl.ds(r, CHUNK)])

    return scat


_scatter128 = _make_scatter(H)


def _mmprep_body(xl_ref, xg_ref, w_ref, degc_ref, hp_ref, dinv_ref):
    for g, x_ref in ((0, xl_ref), (1, xg_ref)):
        d = lax.rsqrt(degc_ref[g] + 1.0)
        dinv_ref[g] = d
        hp_ref[g] = d * jnp.dot(x_ref[...], w_ref[g],
                                preferred_element_type=jnp.float32)


def _stage_body(agg_ref, hp_ref, d_ref, b_ref, w_ref, out_ref):
    for g in range(2):
        d = d_ref[g]
        a = d * (agg_ref[g, :N] + hp_ref[g]) + b_ref[g]
        a = jnp.maximum(a, 0.0)
        out_ref[g] = d * jnp.dot(a, w_ref[g],
                                 preferred_element_type=jnp.float32)


def _stage3_body(agg_ref, hp_ref, d_ref, b_ref, out_ref):
    for g in range(2):
        d = d_ref[g]
        a = d * (agg_ref[g, :N] + hp_ref[g]) + b_ref[g]
        out_ref[g] = d * jnp.maximum(a, 0.0)


def _final_body(agg_ref, hq_ref, d_ref, b_ref, w_ref, outl_ref, outg_ref):
    for g, o_ref in ((0, outl_ref), (1, outg_ref)):
        a = agg_ref[g, :N] + hq_ref[g]
        o_ref[...] = d_ref[g] * jnp.dot(
            a, w_ref[g], preferred_element_type=jnp.float32) + b_ref[g]


def _tc(body, out_shape, *args):
    return pl.pallas_call(body, out_shape=out_shape)(*args)


def kernel(Lnc_f_features, Gene_f_features, Lnc_f_edge_index,
           Gene_f_edge_index, W1l, b1l, W2l, b2l, W3l, b3l,
           W1g, b1g, W2g, b2g, W3g, b3g):
    f32 = jnp.float32
    npad = EP - E
    pad_src = np.asarray((np.arange(npad) * 131) % N, np.int32)
    pad_dst = np.asarray(N + np.arange(npad) % (NP - N), np.int32)
    src = jnp.concatenate([Lnc_f_edge_index[0], pad_src,
                           Gene_f_edge_index[0], pad_src])
    src = src.reshape(ECHUNKS, CHUNK)
    dst = jnp.concatenate([Lnc_f_edge_index[1], pad_dst,
                           Gene_f_edge_index[1], pad_dst])
    dst = dst.reshape(ECHUNKS, CHUNK)
    W1 = jnp.stack([W1l, W1g])
    W2 = jnp.stack([W2l, W2g])
    W3 = jnp.stack([W3l, W3g])
    b1 = jnp.stack([b1l, b1g])[:, None, :]
    b2 = jnp.stack([b2l, b2g])[:, None, :]
    b3 = jnp.stack([b3l, b3g])[:, None, :]
    zeros_h = jnp.zeros((CHUNK, H), f32)

    deg = _deg_kernel(dst).reshape(2, NP)
    degc = deg[:, :N, None]
    hp1, dinvc = _tc(_mmprep_body,
                     (jax.ShapeDtypeStruct((2, N, H), f32),
                      jax.ShapeDtypeStruct((2, N, 1), f32)),
                     Lnc_f_features, Gene_f_features, W1, degc)
    agg1 = _scatter128(hp1, src, dst, zeros_h)
    hp2 = _tc(_stage_body, jax.ShapeDtypeStruct((2, N, H), f32),
              agg1, hp1, dinvc, b1, W2)
    agg2 = _scatter128(hp2, src, dst, zeros_h)
    hq = _tc(_stage3_body, jax.ShapeDtypeStruct((2, N, H), f32),
             agg2, hp2, dinvc, b2)
    agg3 = _scatter128(hq, src, dst, zeros_h)
    return _tc(_final_body, (jax.ShapeDtypeStruct((N, C), f32),
                             jax.ShapeDtypeStruct((N, C), f32)),
               agg3, hq, dinvc, b3, W3)


# speedup vs baseline: 1.0596x; 1.0061x over previous
import functools

import jax
import numpy as np
import jax.numpy as jnp
from jax import lax
from jax.experimental import pallas as pl
from jax.experimental.pallas import tpu as pltpu
from jax.experimental.pallas import tpu_sc as plsc

N = 10000
E = 320000
D = 128
H = 128
C = 16

NTILES = 16
NP = 10240
ROWS_PER_TILE = NP // NTILES
CHUNK = 128
EP = 327680
ECHUNKS = 2 * EP // CHUNK
TILE_ECHUNKS = EP // (NTILES * CHUNK)
BLK = 40
NBLK = TILE_ECHUNKS // BLK

_sc_mesh = plsc.VectorSubcoreMesh(core_axis_name="c", subcore_axis_name="s")


@functools.partial(
    pl.kernel,
    out_type=jax.ShapeDtypeStruct((2 * NP,), jnp.float32),
    mesh=_sc_mesh,
    scratch_types=[
        pltpu.VMEM((CHUNK,), jnp.float32),
        pltpu.VMEM((BLK, CHUNK), jnp.int32),
        pltpu.VMEM((ROWS_PER_TILE,), jnp.float32),
        pltpu.VMEM_SHARED((NP,), jnp.float32),
        pltpu.SemaphoreType.DMA,
    ],
)
def _deg_kernel(dst_hbm, deg_hbm, ones_v, dblk, stage_v, acc_sh, dsem):
    c = lax.axis_index("c")
    s = lax.axis_index("s")
    for j in range(CHUNK // 16):
        ones_v[pl.ds(j * 16, 16)] = jnp.ones((16,), jnp.float32)
    for j in range(ROWS_PER_TILE // 16):
        stage_v[pl.ds(j * 16, 16)] = jnp.zeros((16,), jnp.float32)
    pltpu.sync_copy(stage_v, acc_sh.at[pl.ds(s * ROWS_PER_TILE, ROWS_PER_TILE)])
    plsc.subcore_barrier()

    r0 = c * (ECHUNKS // 2) + s * TILE_ECHUNKS

    def body(i, carry):
        pltpu.sync_copy(dst_hbm.at[pl.ds(r0 + i * BLK, BLK)], dblk)
        descs = [pltpu.async_copy(ones_v, acc_sh.at[dblk.at[j]], dsem,
                                  add=True) for j in range(BLK)]
        for d in descs:
            d.wait()
        return carry

    lax.fori_loop(0, NBLK, body, 0)
    plsc.subcore_barrier()
    pltpu.sync_copy(acc_sh.at[pl.ds(s * ROWS_PER_TILE, ROWS_PER_TILE)], stage_v)
    pltpu.sync_copy(stage_v,
                    deg_hbm.at[pl.ds(c * NP + s * ROWS_PER_TILE, ROWS_PER_TILE)])


def _make_scatter(W):
    nstage = ROWS_PER_TILE // CHUNK

    @functools.partial(
        pl.kernel,
        out_type=jax.ShapeDtypeStruct((2, NP, W), jnp.float32),
        mesh=_sc_mesh,
        scratch_types=[
            pltpu.VMEM((BLK, CHUNK), jnp.int32),
            pltpu.VMEM((BLK, CHUNK), jnp.int32),
            pltpu.VMEM((2, CHUNK, W), jnp.float32),
            pltpu.VMEM_SHARED((NP, W), jnp.float32),
            pltpu.SemaphoreType.DMA,
            pltpu.SemaphoreType.DMA,
        ],
    )
    def scat(hp_hbm, src_hbm, dst_hbm, zeros_hbm, agg_hbm,
             sblk, dblk, rows, acc, gsem, ssem):
        c = lax.axis_index("c")
        s = lax.axis_index("s")
        row0 = s * ROWS_PER_TILE
        pltpu.sync_copy(zeros_hbm, rows.at[0])
        for j in range(nstage):
            pltpu.sync_copy(rows.at[0], acc.at[pl.ds(row0 + j * CHUNK, CHUNK)])
        plsc.subcore_barrier()

        r0 = c * (ECHUNKS // 2) + s * TILE_ECHUNKS

        def body(i, carry):
            pltpu.sync_copy(src_hbm.at[pl.ds(r0 + i * BLK, BLK)], sblk)
            pltpu.sync_copy(dst_hbm.at[pl.ds(r0 + i * BLK, BLK)], dblk)
            desc = pltpu.async_copy(hp_hbm.at[c].at[sblk.at[0]],
                                    rows.at[0], gsem)
            for j in range(BLK):
                b = j & 1
                if j + 1 < BLK:
                    nxt = pltpu.async_copy(hp_hbm.at[c].at[sblk.at[j + 1]],
                                           rows.at[b ^ 1], gsem)
                desc.wait()
                pltpu.sync_copy(rows.at[b], acc.at[dblk.at[j]], add=True)
                if j + 1 < BLK:
                    desc = nxt
            return carry

        lax.fori_loop(0, NBLK, body, 0)
        plsc.subcore_barrier()
        for j in range(nstage):
            r = row0 + j * CHUNK
            pltpu.sync_copy(acc.at[pl.ds(r, CHUNK)], rows.at[0])
            pltpu.sync_copy(rows.at[0], agg_hbm.at[c, pl.ds(r, CHUNK)])

    return scat


_scatter128 = _make_scatter(H)


def _mmprep_body(xl_ref, xg_ref, w_ref, degc_ref, hp_ref, dinv_ref):
    for g, x_ref in ((0, xl_ref), (1, xg_ref)):
        d = lax.rsqrt(degc_ref[g] + 1.0)
        dinv_ref[g] = d
        hp_ref[g] = d * jnp.dot(x_ref[...], w_ref[g],
                                preferred_element_type=jnp.float32)


def _stage_body(agg_ref, hp_ref, d_ref, b_ref, w_ref, out_ref):
    for g in range(2):
        d = d_ref[g]
        a = d * (agg_ref[g, :N] + hp_ref[g]) + b_ref[g]
        a = jnp.maximum(a, 0.0)
        out_ref[g] = d * jnp.dot(a, w_ref[g],
                                 preferred_element_type=jnp.float32)


def _stage3_body(agg_ref, hp_ref, d_ref, b_ref, out_ref):
    for g in range(2):
        d = d_ref[g]
        a = d * (agg_ref[g, :N] + hp_ref[g]) + b_ref[g]
        out_ref[g] = d * jnp.maximum(a, 0.0)


def _final_body(agg_ref, hq_ref, d_ref, b_ref, w_ref, outl_ref, outg_ref):
    for g, o_ref in ((0, outl_ref), (1, outg_ref)):
        a = agg_ref[g, :N] + hq_ref[g]
        o_ref[...] = d_ref[g] * jnp.dot(
            a, w_ref[g], preferred_element_type=jnp.float32) + b_ref[g]


def _tc(body, out_shape, *args):
    return pl.pallas_call(body, out_shape=out_shape)(*args)


def kernel(Lnc_f_features, Gene_f_features, Lnc_f_edge_index,
           Gene_f_edge_index, W1l, b1l, W2l, b2l, W3l, b3l,
           W1g, b1g, W2g, b2g, W3g, b3g):
    f32 = jnp.float32
    npad = EP - E
    pad_src = np.asarray((np.arange(npad) * 131) % N, np.int32)
    pad_dst = np.asarray(N + np.arange(npad) % (NP - N), np.int32)
    src = jnp.concatenate([Lnc_f_edge_index[0], pad_src,
                           Gene_f_edge_index[0], pad_src])
    src = src.reshape(ECHUNKS, CHUNK)
    dst = jnp.concatenate([Lnc_f_edge_index[1], pad_dst,
                           Gene_f_edge_index[1], pad_dst])
    dst = dst.reshape(ECHUNKS, CHUNK)
    W1 = jnp.stack([W1l, W1g])
    W2 = jnp.stack([W2l, W2g])
    W3 = jnp.stack([W3l, W3g])
    b1 = jnp.stack([b1l, b1g])[:, None, :]
    b2 = jnp.stack([b2l, b2g])[:, None, :]
    b3 = jnp.stack([b3l, b3g])[:, None, :]
    zeros_h = jnp.zeros((CHUNK, H), f32)

    deg = _deg_kernel(dst).reshape(2, NP)
    degc = deg[:, :N, None]
    hp1, dinvc = _tc(_mmprep_body,
                     (jax.ShapeDtypeStruct((2, N, H), f32),
                      jax.ShapeDtypeStruct((2, N, 1), f32)),
                     Lnc_f_features, Gene_f_features, W1, degc)
    agg1 = _scatter128(hp1, src, dst, zeros_h)
    hp2 = _tc(_stage_body, jax.ShapeDtypeStruct((2, N, H), f32),
              agg1, hp1, dinvc, b1, W2)
    agg2 = _scatter128(hp2, src, dst, zeros_h)
    hq = _tc(_stage3_body, jax.ShapeDtypeStruct((2, N, H), f32),
             agg2, hp2, dinvc, b2)
    agg3 = _scatter128(hq, src, dst, zeros_h)
    return _tc(_final_body, (jax.ShapeDtypeStruct((N, C), f32),
                             jax.ShapeDtypeStruct((N, C), f32)),
               agg3, hq, dinvc, b3, W3)


# speedup vs baseline: 1.0716x; 1.0113x over previous
import functools

import jax
import numpy as np
import jax.numpy as jnp
from jax import lax
from jax.experimental import pallas as pl
from jax.experimental.pallas import tpu as pltpu
from jax.experimental.pallas import tpu_sc as plsc

N = 10000
E = 320000
D = 128
H = 128
C = 16

NTILES = 16
NP = 10240
ROWS_PER_TILE = NP // NTILES
CHUNK = 128
EP = 327680
ECHUNKS = 2 * EP // CHUNK
TILE_ECHUNKS = EP // (NTILES * CHUNK)
BLK = 40
NBLK = TILE_ECHUNKS // BLK

_sc_mesh = plsc.VectorSubcoreMesh(core_axis_name="c", subcore_axis_name="s")


@functools.partial(
    pl.kernel,
    out_type=jax.ShapeDtypeStruct((2 * NP,), jnp.float32),
    mesh=_sc_mesh,
    scratch_types=[
        pltpu.VMEM((CHUNK,), jnp.float32),
        pltpu.VMEM((BLK, CHUNK), jnp.int32),
        pltpu.VMEM((ROWS_PER_TILE,), jnp.float32),
        pltpu.VMEM_SHARED((NP,), jnp.float32),
        pltpu.SemaphoreType.DMA,
    ],
)
def _deg_kernel(dst_hbm, deg_hbm, ones_v, dblk, stage_v, acc_sh, dsem):
    c = lax.axis_index("c")
    s = lax.axis_index("s")
    for j in range(CHUNK // 16):
        ones_v[pl.ds(j * 16, 16)] = jnp.ones((16,), jnp.float32)
    for j in range(ROWS_PER_TILE // 16):
        stage_v[pl.ds(j * 16, 16)] = jnp.zeros((16,), jnp.float32)
    pltpu.sync_copy(stage_v, acc_sh.at[pl.ds(s * ROWS_PER_TILE, ROWS_PER_TILE)])
    plsc.subcore_barrier()

    r0 = c * (ECHUNKS // 2) + s * TILE_ECHUNKS

    def body(i, carry):
        pltpu.sync_copy(dst_hbm.at[pl.ds(r0 + i * BLK, BLK)], dblk)
        descs = [pltpu.async_copy(ones_v, acc_sh.at[dblk.at[j]], dsem,
                                  add=True) for j in range(BLK)]
        for d in descs:
            d.wait()
        return carry

    lax.fori_loop(0, NBLK, body, 0)
    plsc.subcore_barrier()
    pltpu.sync_copy(acc_sh.at[pl.ds(s * ROWS_PER_TILE, ROWS_PER_TILE)], stage_v)
    pltpu.sync_copy(stage_v,
                    deg_hbm.at[pl.ds(c * NP + s * ROWS_PER_TILE, ROWS_PER_TILE)])


def _make_scatter(W):
    nstage = ROWS_PER_TILE // CHUNK

    @functools.partial(
        pl.kernel,
        out_type=jax.ShapeDtypeStruct((2, NP, W), jnp.float32),
        mesh=_sc_mesh,
        scratch_types=[
            pltpu.VMEM((BLK, CHUNK), jnp.int32),
            pltpu.VMEM((BLK, CHUNK), jnp.int32),
            pltpu.VMEM((2, CHUNK, W), jnp.float32),
            pltpu.VMEM_SHARED((NP, W), jnp.float32),
            pltpu.SemaphoreType.DMA,
            pltpu.SemaphoreType.DMA,
        ],
    )
    def scat(hp_hbm, src_hbm, dst_hbm, zeros_hbm, agg_hbm,
             sblk, dblk, rows, acc, gsem, ssem):
        c = lax.axis_index("c")
        s = lax.axis_index("s")
        row0 = s * ROWS_PER_TILE
        pltpu.sync_copy(zeros_hbm, rows.at[0])
        zdescs = [pltpu.async_copy(rows.at[0],
                                   acc.at[pl.ds(row0 + j * CHUNK, CHUNK)],
                                   ssem) for j in range(nstage)]
        for d in zdescs:
            d.wait()
        plsc.subcore_barrier()

        r0 = c * (ECHUNKS // 2) + s * TILE_ECHUNKS

        def body(i, carry):
            pltpu.sync_copy(src_hbm.at[pl.ds(r0 + i * BLK, BLK)], sblk)
            pltpu.sync_copy(dst_hbm.at[pl.ds(r0 + i * BLK, BLK)], dblk)
            desc = pltpu.async_copy(hp_hbm.at[c].at[sblk.at[0]],
                                    rows.at[0], gsem)
            for j in range(BLK):
                b = j & 1
                if j + 1 < BLK:
                    nxt = pltpu.async_copy(hp_hbm.at[c].at[sblk.at[j + 1]],
                                           rows.at[b ^ 1], gsem)
                desc.wait()
                pltpu.sync_copy(rows.at[b], acc.at[dblk.at[j]], add=True)
                if j + 1 < BLK:
                    desc = nxt
            return carry

        lax.fori_loop(0, NBLK, body, 0)
        plsc.subcore_barrier()
        odescs = []
        for j in range(nstage):
            b = j & 1
            if j >= 2:
                odescs[j - 2].wait()
            r = row0 + j * CHUNK
            pltpu.sync_copy(acc.at[pl.ds(r, CHUNK)], rows.at[b])
            odescs.append(pltpu.async_copy(
                rows.at[b], agg_hbm.at[c, pl.ds(r, CHUNK)], gsem))
        odescs[nstage - 2].wait()
        odescs[nstage - 1].wait()

    return scat


_scatter128 = _make_scatter(H)


def _mmprep_body(xl_ref, xg_ref, w_ref, degc_ref, hp_ref, dinv_ref):
    for g, x_ref in ((0, xl_ref), (1, xg_ref)):
        d = lax.rsqrt(degc_ref[g] + 1.0)
        dinv_ref[g] = d
        hp_ref[g] = d * jnp.dot(x_ref[...], w_ref[g],
                                preferred_element_type=jnp.float32)


def _stage_body(agg_ref, hp_ref, d_ref, b_ref, w_ref, out_ref):
    for g in range(2):
        d = d_ref[g]
        a = d * (agg_ref[g, :N] + hp_ref[g]) + b_ref[g]
        a = jnp.maximum(a, 0.0)
        out_ref[g] = d * jnp.dot(a, w_ref[g],
                                 preferred_element_type=jnp.float32)


def _stage3_body(agg_ref, hp_ref, d_ref, b_ref, out_ref):
    for g in range(2):
        d = d_ref[g]
        a = d * (agg_ref[g, :N] + hp_ref[g]) + b_ref[g]
        out_ref[g] = d * jnp.maximum(a, 0.0)


def _final_body(agg_ref, hq_ref, d_ref, b_ref, w_ref, outl_ref, outg_ref):
    for g, o_ref in ((0, outl_ref), (1, outg_ref)):
        a = agg_ref[g, :N] + hq_ref[g]
        o_ref[...] = d_ref[g] * jnp.dot(
            a, w_ref[g], preferred_element_type=jnp.float32) + b_ref[g]


def _tc(body, out_shape, *args):
    return pl.pallas_call(body, out_shape=out_shape)(*args)


def kernel(Lnc_f_features, Gene_f_features, Lnc_f_edge_index,
           Gene_f_edge_index, W1l, b1l, W2l, b2l, W3l, b3l,
           W1g, b1g, W2g, b2g, W3g, b3g):
    f32 = jnp.float32
    npad = EP - E
    pad_src = np.asarray((np.arange(npad) * 131) % N, np.int32)
    pad_dst = np.asarray(N + np.arange(npad) % (NP - N), np.int32)
    src = jnp.concatenate([Lnc_f_edge_index[0], pad_src,
                           Gene_f_edge_index[0], pad_src])
    src = src.reshape(ECHUNKS, CHUNK)
    dst = jnp.concatenate([Lnc_f_edge_index[1], pad_dst,
                           Gene_f_edge_index[1], pad_dst])
    dst = dst.reshape(ECHUNKS, CHUNK)
    W1 = jnp.stack([W1l, W1g])
    W2 = jnp.stack([W2l, W2g])
    W3 = jnp.stack([W3l, W3g])
    b1 = jnp.stack([b1l, b1g])[:, None, :]
    b2 = jnp.stack([b2l, b2g])[:, None, :]
    b3 = jnp.stack([b3l, b3g])[:, None, :]
    zeros_h = jnp.zeros((CHUNK, H), f32)

    deg = _deg_kernel(dst).reshape(2, NP)
    degc = deg[:, :N, None]
    hp1, dinvc = _tc(_mmprep_body,
                     (jax.ShapeDtypeStruct((2, N, H), f32),
                      jax.ShapeDtypeStruct((2, N, 1), f32)),
                     Lnc_f_features, Gene_f_features, W1, degc)
    agg1 = _scatter128(hp1, src, dst, zeros_h)
    hp2 = _tc(_stage_body, jax.ShapeDtypeStruct((2, N, H), f32),
              agg1, hp1, dinvc, b1, W2)
    agg2 = _scatter128(hp2, src, dst, zeros_h)
    hq = _tc(_stage3_body, jax.ShapeDtypeStruct((2, N, H), f32),
             agg2, hp2, dinvc, b2)
    agg3 = _scatter128(hq, src, dst, zeros_h)
    return _tc(_final_body, (jax.ShapeDtypeStruct((N, C), f32),
                             jax.ShapeDtypeStruct((N, C), f32)),
               agg3, hq, dinvc, b3, W3)
